# baseline (speedup 1.0000x reference)
import functools

import jax
import jax.numpy as jnp
from jax.experimental import pallas as pl
from jax.experimental.pallas import tpu as pltpu

_L = 4
_B = 8
_T = 2048
_D = 256
_TTL = 1024
_N = _B * _T
_HALF_LIFE = 256.0
_BIG = (1 << 30)
_CPB = 8


def _prep(scal_ref, p_ref, bs_ref, w_scr, wsn_scr, ls_scr):
    step = scal_ref[0]
    horizon = scal_ref[1]
    k = scal_ref[2]

    p = p_ref[...]
    bits = jax.lax.bitcast_convert_type(p, jnp.int32)
    s = bits ^ jax.lax.shift_right_arithmetic(bits, 31).astype(jnp.int32) & jnp.int32(0x7FFFFFFF)

    t = jnp.int32(-2147483648)
    for bit in range(30, -1, -1):
        tp = t + jnp.int32(1 << bit)
        cnt = jnp.sum((s >= tp).astype(jnp.int32))
        t = jnp.where(cnt >= k, tp, t)

    c_gt = jnp.sum((s > t).astype(jnp.int32))
    need = k - c_gt
    eq = s == t
    row = jax.lax.broadcasted_iota(jnp.int32, (_B, _T), 0)
    col = jax.lax.broadcasted_iota(jnp.int32, (_B, _T), 1)
    fi = row * _T + col
    sel_i = jnp.int32(0)
    for bit in range(13, -1, -1):
        ip = sel_i | jnp.int32(1 << bit)
        f = jnp.sum((eq & (fi <= ip)).astype(jnp.int32))
        sel_i = jnp.where(f <= need, ip, sel_i)

    mask = (s > t) | (eq & (fi <= sel_i))
    mf = mask.astype(jnp.float32)
    n_ev = jnp.sum(mf)
    inv_n = 1.0 / jnp.maximum(n_ev, 1.0)

    cntw = jnp.minimum(horizon, (_T - 1) - col)
    g = jnp.where(cntw > 0, mf / jnp.maximum(cntw, 1).astype(jnp.float32), 0.0)

    def shr(x, n):
        if n >= _T:
            return jnp.zeros_like(x)
        return jnp.concatenate([jnp.zeros((_B, n), dtype=x.dtype), x[:, :-n]], axis=1)

    big_g = g
    sh = 1
    while sh < _T:
        big_g = big_g + shr(big_g, sh)
        sh *= 2
    hp1 = jnp.minimum(horizon + 1, _T + 1)
    shifted = big_g
    for bit in range(12):
        amt = 1 << bit
        cond = ((hp1 >> bit) & 1) == 1
        shifted = jnp.where(cond, shr(shifted, amt), shifted)
    w_scr[...] = shr(big_g, 1) - shifted

    bsv = bs_ref[...]
    sidx = jax.lax.broadcasted_iota(jnp.int32, (_L, _TTL), 1)
    empty = bsv == jnp.int32(-1)
    first_empty = jnp.min(jnp.where(empty, sidx, _BIG), axis=1, keepdims=True)
    minval = jnp.min(bsv, axis=1, keepdims=True)
    first_min = jnp.min(jnp.where(bsv == minval, sidx, _BIG), axis=1, keepdims=True)
    slot = jnp.where(first_empty < _BIG, first_empty, first_min)

    age = jnp.maximum(step - bsv, 0).astype(jnp.float32)
    wgt = jnp.exp2(-age / _HALF_LIFE) * (bsv >= 0).astype(jnp.float32)
    wgt = jnp.where(sidx == slot, 0.0, wgt)
    wsum = jnp.sum(wgt, axis=1, keepdims=True) + 1.0
    wsn_scr[...] = wgt / wsum
    ls_scr[...] = inv_n / wsum


def _fused_kernel(scal_ref, p_ref, bs_ref, x_ref, be_ref, out_ref,
                  w_scr, wsn_scr, ls_scr):
    l = pl.program_id(0)
    b = pl.program_id(1)

    @pl.when((l == 0) & (b == 0))
    def _():
        _prep(scal_ref, p_ref, bs_ref, w_scr, wsn_scr, ls_scr)

    part = jnp.zeros((1, _D), jnp.float32)
    for c in range(_CPB):
        x = x_ref[0, c]
        wrow = w_scr[pl.ds(b * _CPB + c, 1), :]
        part += jax.lax.dot_general(
            wrow, x * x, (((1,), (0,)), ((), ())),
            preferred_element_type=jnp.float32)
    contrib = (ls_scr[pl.ds(l, 1), :] * part)[None]

    @pl.when(b == 0)
    def _():
        be = be_ref[0]
        wsrow = wsn_scr[pl.ds(l, 1), :]
        bank = jax.lax.dot_general(
            wsrow, be, (((1,), (0,)), ((), ())),
            preferred_element_type=jnp.float32)
        out_ref[...] = bank[None] + contrib

    @pl.when(b != 0)
    def _():
        out_ref[...] += contrib


@jax.jit
def kernel(pressure, states, bank_evidence, bank_step, bank_event_count,
           baseline_future_energy, step, horizon_H, events_k):
    del bank_event_count, baseline_future_energy
    scal = jnp.stack([jnp.asarray(step, jnp.int32),
                      jnp.asarray(horizon_H, jnp.int32),
                      jnp.asarray(events_k, jnp.int32)])

    score = pl.pallas_call(
        _fused_kernel,
        grid=(_L, _B // _CPB),
        in_specs=[
            pl.BlockSpec(memory_space=pltpu.MemorySpace.SMEM),
            pl.BlockSpec((_B, _T), lambda l, b: (0, 0)),
            pl.BlockSpec((_L, _TTL), lambda l, b: (0, 0)),
            pl.BlockSpec((1, _CPB, _T, _D), lambda l, b: (l, b, 0, 0)),
            pl.BlockSpec((1, _TTL, _D), lambda l, b: (l, 0, 0)),
        ],
        out_specs=pl.BlockSpec((1, 1, _D), lambda l, b: (l, 0, 0)),
        out_shape=jax.ShapeDtypeStruct((_L, 1, _D), jnp.float32),
        scratch_shapes=[
            pltpu.VMEM((_B, _T), jnp.float32),
            pltpu.VMEM((_L, _TTL), jnp.float32),
            pltpu.VMEM((_L, 1), jnp.float32),
        ],
        compiler_params=pltpu.CompilerParams(
            dimension_semantics=("arbitrary", "arbitrary")),
    )(scal, pressure, bank_step, states, bank_evidence)

    return score.reshape(_L, _D)


# speedup vs baseline: 1.0820x; 1.0820x over previous
import functools

import jax
import jax.numpy as jnp
from jax.experimental import pallas as pl
from jax.experimental.pallas import tpu as pltpu

_L = 4
_B = 8
_T = 2048
_D = 256
_TTL = 1024
_N = _B * _T
_HALF_LIFE = 256.0
_BIG = (1 << 30)
_CPB = 4


def _prep(scal_ref, p_ref, bs_ref, w_scr, wsn_scr, ls_scr):
    step = scal_ref[0]
    horizon = scal_ref[1]
    k = scal_ref[2]

    p = p_ref[...]
    bits = jax.lax.bitcast_convert_type(p, jnp.int32)
    s = bits ^ jax.lax.shift_right_arithmetic(bits, 31).astype(jnp.int32) & jnp.int32(0x7FFFFFFF)

    t = jnp.int32(-2147483648)
    tp = t + jnp.int32(1 << 30)
    t = jnp.where(jnp.sum((s >= tp).astype(jnp.int32)) >= k, tp, t)
    for p in range(28, -2, -2):
        c1 = jnp.sum((s >= t + jnp.int32(1 << p)).astype(jnp.int32))
        c2 = jnp.sum((s >= t + jnp.int32(2 << p)).astype(jnp.int32))
        c3 = jnp.sum((s >= t + jnp.int32(3 << p)).astype(jnp.int32))
        t = t + jnp.where(c3 >= k, jnp.int32(3 << p),
                          jnp.where(c2 >= k, jnp.int32(2 << p),
                                    jnp.where(c1 >= k, jnp.int32(1 << p), 0)))

    c_gt = jnp.sum((s > t).astype(jnp.int32))
    need = k - c_gt
    eq = s == t
    row = jax.lax.broadcasted_iota(jnp.int32, (_B, _T), 0)
    col = jax.lax.broadcasted_iota(jnp.int32, (_B, _T), 1)
    fi = row * _T + col
    sel_i = jnp.int32(0)
    for p in range(12, -2, -2):
        o1 = sel_i | jnp.int32(1 << p)
        o2 = sel_i | jnp.int32(2 << p)
        o3 = sel_i | jnp.int32(3 << p)
        f1 = jnp.sum((eq & (fi <= o1)).astype(jnp.int32))
        f2 = jnp.sum((eq & (fi <= o2)).astype(jnp.int32))
        f3 = jnp.sum((eq & (fi <= o3)).astype(jnp.int32))
        sel_i = jnp.where(f3 <= need, o3,
                          jnp.where(f2 <= need, o2,
                                    jnp.where(f1 <= need, o1, sel_i)))

    mask = (s > t) | (eq & (fi <= sel_i))
    mf = mask.astype(jnp.float32)
    n_ev = jnp.sum(mf)
    inv_n = 1.0 / jnp.maximum(n_ev, 1.0)

    cntw = jnp.minimum(horizon, (_T - 1) - col)
    g = jnp.where(cntw > 0, mf / jnp.maximum(cntw, 1).astype(jnp.float32), 0.0)

    def shr(x, n):
        if n >= _T:
            return jnp.zeros_like(x)
        return jnp.concatenate([jnp.zeros((_B, n), dtype=x.dtype), x[:, :-n]], axis=1)

    big_g = g
    sh = 1
    while sh < _T:
        big_g = big_g + shr(big_g, sh)
        sh *= 2
    hp1 = jnp.minimum(horizon + 1, _T + 1)
    shifted = big_g
    for bit in range(12):
        amt = 1 << bit
        cond = ((hp1 >> bit) & 1) == 1
        shifted = jnp.where(cond, shr(shifted, amt), shifted)
    w_scr[...] = shr(big_g, 1) - shifted

    bsv = bs_ref[...]
    sidx = jax.lax.broadcasted_iota(jnp.int32, (_L, _TTL), 1)
    empty = bsv == jnp.int32(-1)
    first_empty = jnp.min(jnp.where(empty, sidx, _BIG), axis=1, keepdims=True)
    minval = jnp.min(bsv, axis=1, keepdims=True)
    first_min = jnp.min(jnp.where(bsv == minval, sidx, _BIG), axis=1, keepdims=True)
    slot = jnp.where(first_empty < _BIG, first_empty, first_min)

    age = jnp.maximum(step - bsv, 0).astype(jnp.float32)
    wgt = jnp.exp2(-age / _HALF_LIFE) * (bsv >= 0).astype(jnp.float32)
    wgt = jnp.where(sidx == slot, 0.0, wgt)
    wsum = jnp.sum(wgt, axis=1, keepdims=True) + 1.0
    wsn_scr[...] = wgt / wsum
    ls_scr[...] = inv_n / wsum


def _fused_kernel(scal_ref, p_ref, bs_ref, x_ref, be_ref, out_ref,
                  w_scr, wsn_scr, ls_scr):
    l = pl.program_id(0)
    b = pl.program_id(1)

    @pl.when((l == 0) & (b == 0))
    def _():
        _prep(scal_ref, p_ref, bs_ref, w_scr, wsn_scr, ls_scr)

    part = jnp.zeros((1, _D), jnp.float32)
    for c in range(_CPB):
        x = x_ref[0, c]
        wrow = w_scr[pl.ds(b * _CPB + c, 1), :]
        part += jax.lax.dot_general(
            wrow, x * x, (((1,), (0,)), ((), ())),
            preferred_element_type=jnp.float32)
    contrib = (ls_scr[pl.ds(l, 1), :] * part)[None]

    @pl.when(b == 0)
    def _():
        be = be_ref[0]
        wsrow = wsn_scr[pl.ds(l, 1), :]
        bank = jax.lax.dot_general(
            wsrow, be, (((1,), (0,)), ((), ())),
            preferred_element_type=jnp.float32)
        out_ref[...] = bank[None] + contrib

    @pl.when(b != 0)
    def _():
        out_ref[...] += contrib


@jax.jit
def kernel(pressure, states, bank_evidence, bank_step, bank_event_count,
           baseline_future_energy, step, horizon_H, events_k):
    del bank_event_count, baseline_future_energy
    scal = jnp.stack([jnp.asarray(step, jnp.int32),
                      jnp.asarray(horizon_H, jnp.int32),
                      jnp.asarray(events_k, jnp.int32)])

    score = pl.pallas_call(
        _fused_kernel,
        grid=(_L, _B // _CPB),
        in_specs=[
            pl.BlockSpec(memory_space=pltpu.MemorySpace.SMEM),
            pl.BlockSpec((_B, _T), lambda l, b: (0, 0)),
            pl.BlockSpec((_L, _TTL), lambda l, b: (0, 0)),
            pl.BlockSpec((1, _CPB, _T, _D), lambda l, b: (l, b, 0, 0)),
            pl.BlockSpec((1, _TTL, _D), lambda l, b: (l, 0, 0)),
        ],
        out_specs=pl.BlockSpec((1, 1, _D), lambda l, b: (l, 0, 0)),
        out_shape=jax.ShapeDtypeStruct((_L, 1, _D), jnp.float32),
        scratch_shapes=[
            pltpu.VMEM((_B, _T), jnp.float32),
            pltpu.VMEM((_L, _TTL), jnp.float32),
            pltpu.VMEM((_L, 1), jnp.float32),
        ],
        compiler_params=pltpu.CompilerParams(
            dimension_semantics=("arbitrary", "arbitrary")),
    )(scal, pressure, bank_step, states, bank_evidence)

    return score.reshape(_L, _D)


# speedup vs baseline: 1.1102x; 1.0260x over previous
import functools

import jax
import jax.numpy as jnp
from jax.experimental import pallas as pl
from jax.experimental.pallas import tpu as pltpu

_L = 4
_B = 8
_T = 2048
_D = 256
_TTL = 1024
_N = _B * _T
_HALF_LIFE = 256.0
_BIG = (1 << 30)
_CPB = 4


def _prep(scal_ref, p_ref, bs_ref, w_scr, wsn_scr, ls_scr):
    step = scal_ref[0]
    horizon = scal_ref[1]
    k = scal_ref[2]

    p = p_ref[...]
    bits = jax.lax.bitcast_convert_type(p, jnp.int32)
    s = bits ^ jax.lax.shift_right_arithmetic(bits, 31).astype(jnp.int32) & jnp.int32(0x7FFFFFFF)

    t = jnp.int32(-2147483648)
    tp = t + jnp.int32(1 << 30)
    t = jnp.where(jnp.sum((s >= tp).astype(jnp.int32)) >= k, tp, t)
    for p in range(27, -3, -3):
        cs = [jnp.sum((s >= t + jnp.int32(q << p)).astype(jnp.int32))
              for q in range(1, 8)]
        inc = jnp.int32(0)
        for q in range(1, 8):
            inc = jnp.where(cs[q - 1] >= k, jnp.int32(q << p), inc)
        t = t + inc

    c_gt = jnp.sum((s > t).astype(jnp.int32))
    need = k - c_gt
    eq = s == t
    row = jax.lax.broadcasted_iota(jnp.int32, (_B, _T), 0)
    col = jax.lax.broadcasted_iota(jnp.int32, (_B, _T), 1)
    fi = row * _T + col
    def _tie_search(_):
        sel = jnp.int32(0)
        for p in range(12, -2, -2):
            o1 = sel | jnp.int32(1 << p)
            o2 = sel | jnp.int32(2 << p)
            o3 = sel | jnp.int32(3 << p)
            f1 = jnp.sum((eq & (fi <= o1)).astype(jnp.int32))
            f2 = jnp.sum((eq & (fi <= o2)).astype(jnp.int32))
            f3 = jnp.sum((eq & (fi <= o3)).astype(jnp.int32))
            sel = jnp.where(f3 <= need, o3,
                            jnp.where(f2 <= need, o2,
                                      jnp.where(f1 <= need, o1, sel)))
        return sel

    c_eq = jnp.sum(eq.astype(jnp.int32))
    sel_i = jax.lax.cond(need == c_eq,
                         lambda _: jnp.int32(_N - 1), _tie_search, None)

    mask = (s > t) | (eq & (fi <= sel_i))
    mf = mask.astype(jnp.float32)
    n_ev = jnp.sum(mf)
    inv_n = 1.0 / jnp.maximum(n_ev, 1.0)

    cntw = jnp.minimum(horizon, (_T - 1) - col)
    g = jnp.where(cntw > 0, mf / jnp.maximum(cntw, 1).astype(jnp.float32), 0.0)

    def shr(x, n):
        if n >= _T:
            return jnp.zeros_like(x)
        return jnp.concatenate([jnp.zeros((_B, n), dtype=x.dtype), x[:, :-n]], axis=1)

    big_g = g
    sh = 1
    while sh < _T:
        big_g = big_g + shr(big_g, sh)
        sh *= 2
    hp1 = jnp.minimum(horizon + 1, _T + 1)
    shifted = big_g
    for bit in range(12):
        amt = 1 << bit
        cond = ((hp1 >> bit) & 1) == 1
        shifted = jnp.where(cond, shr(shifted, amt), shifted)
    w_scr[...] = shr(big_g, 1) - shifted

    bsv = bs_ref[...]
    sidx = jax.lax.broadcasted_iota(jnp.int32, (_L, _TTL), 1)
    empty = bsv == jnp.int32(-1)
    first_empty = jnp.min(jnp.where(empty, sidx, _BIG), axis=1, keepdims=True)
    minval = jnp.min(bsv, axis=1, keepdims=True)
    first_min = jnp.min(jnp.where(bsv == minval, sidx, _BIG), axis=1, keepdims=True)
    slot = jnp.where(first_empty < _BIG, first_empty, first_min)

    age = jnp.maximum(step - bsv, 0).astype(jnp.float32)
    wgt = jnp.exp2(-age / _HALF_LIFE) * (bsv >= 0).astype(jnp.float32)
    wgt = jnp.where(sidx == slot, 0.0, wgt)
    wsum = jnp.sum(wgt, axis=1, keepdims=True) + 1.0
    wsn_scr[...] = wgt / wsum
    ls_scr[...] = inv_n / wsum


def _fused_kernel(scal_ref, p_ref, bs_ref, x_ref, be_ref, out_ref,
                  w_scr, wsn_scr, ls_scr):
    l = pl.program_id(0)
    b = pl.program_id(1)

    @pl.when((l == 0) & (b == 0))
    def _():
        _prep(scal_ref, p_ref, bs_ref, w_scr, wsn_scr, ls_scr)

    part = jnp.zeros((1, _D), jnp.float32)
    for c in range(_CPB):
        x = x_ref[0, c]
        wrow = w_scr[pl.ds(b * _CPB + c, 1), :]
        part += jax.lax.dot_general(
            wrow, x * x, (((1,), (0,)), ((), ())),
            preferred_element_type=jnp.float32)
    contrib = (ls_scr[pl.ds(l, 1), :] * part)[None]

    @pl.when(b == 0)
    def _():
        be = be_ref[0]
        wsrow = wsn_scr[pl.ds(l, 1), :]
        bank = jax.lax.dot_general(
            wsrow, be, (((1,), (0,)), ((), ())),
            preferred_element_type=jnp.float32)
        out_ref[...] = bank[None] + contrib

    @pl.when(b != 0)
    def _():
        out_ref[...] += contrib


@jax.jit
def kernel(pressure, states, bank_evidence, bank_step, bank_event_count,
           baseline_future_energy, step, horizon_H, events_k):
    del bank_event_count, baseline_future_energy
    scal = jnp.stack([jnp.asarray(step, jnp.int32),
                      jnp.asarray(horizon_H, jnp.int32),
                      jnp.asarray(events_k, jnp.int32)])

    score = pl.pallas_call(
        _fused_kernel,
        grid=(_L, _B // _CPB),
        in_specs=[
            pl.BlockSpec(memory_space=pltpu.MemorySpace.SMEM),
            pl.BlockSpec((_B, _T), lambda l, b: (0, 0)),
            pl.BlockSpec((_L, _TTL), lambda l, b: (0, 0)),
            pl.BlockSpec((1, _CPB, _T, _D), lambda l, b: (l, b, 0, 0)),
            pl.BlockSpec((1, _TTL, _D), lambda l, b: (l, 0, 0)),
        ],
        out_specs=pl.BlockSpec((1, 1, _D), lambda l, b: (l, 0, 0)),
        out_shape=jax.ShapeDtypeStruct((_L, 1, _D), jnp.float32),
        scratch_shapes=[
            pltpu.VMEM((_B, _T), jnp.float32),
            pltpu.VMEM((_L, _TTL), jnp.float32),
            pltpu.VMEM((_L, 1), jnp.float32),
        ],
        compiler_params=pltpu.CompilerParams(
            dimension_semantics=("arbitrary", "arbitrary")),
    )(scal, pressure, bank_step, states, bank_evidence)

    return score.reshape(_L, _D)


# speedup vs baseline: 1.2242x; 1.1027x over previous
import functools

import jax
import jax.numpy as jnp
from jax.experimental import pallas as pl
from jax.experimental.pallas import tpu as pltpu

_L = 4
_B = 8
_T = 2048
_D = 256
_TTL = 1024
_N = _B * _T
_HALF_LIFE = 256.0
_BIG = (1 << 30)
_CPB = 4


def _prep(scal_ref, p_ref, bs_ref, w_scr, wsn_scr, ls_scr):
    step = scal_ref[0]
    horizon = scal_ref[1]
    k = scal_ref[2]

    p = p_ref[...]
    bits = jax.lax.bitcast_convert_type(p, jnp.int32)
    s = bits ^ jax.lax.shift_right_arithmetic(bits, 31).astype(jnp.int32) & jnp.int32(0x7FFFFFFF)

    t = jnp.int32(-2147483648)
    tp = t + jnp.int32(1 << 30)
    t = jnp.where(jnp.sum((s >= tp).astype(jnp.int32)) >= k, tp, t)
    for p in range(27, -3, -3):
        cs = [jnp.sum((s >= t + jnp.int32(q << p)).astype(jnp.int32))
              for q in range(1, 8)]
        inc = jnp.int32(0)
        for q in range(1, 8):
            inc = jnp.where(cs[q - 1] >= k, jnp.int32(q << p), inc)
        t = t + inc

    c_gt = jnp.sum((s > t).astype(jnp.int32))
    need = k - c_gt
    eq = s == t
    row = jax.lax.broadcasted_iota(jnp.int32, (_B, _T), 0)
    col = jax.lax.broadcasted_iota(jnp.int32, (_B, _T), 1)
    fi = row * _T + col
    def _tie_search(_):
        sel = jnp.int32(0)
        for p in range(12, -2, -2):
            o1 = sel | jnp.int32(1 << p)
            o2 = sel | jnp.int32(2 << p)
            o3 = sel | jnp.int32(3 << p)
            f1 = jnp.sum((eq & (fi <= o1)).astype(jnp.int32))
            f2 = jnp.sum((eq & (fi <= o2)).astype(jnp.int32))
            f3 = jnp.sum((eq & (fi <= o3)).astype(jnp.int32))
            sel = jnp.where(f3 <= need, o3,
                            jnp.where(f2 <= need, o2,
                                      jnp.where(f1 <= need, o1, sel)))
        return sel

    c_eq = jnp.sum(eq.astype(jnp.int32))
    sel_i = jax.lax.cond(need == c_eq,
                         lambda _: jnp.int32(_N - 1), _tie_search, None)

    mask = (s > t) | (eq & (fi <= sel_i))
    mf = mask.astype(jnp.float32)
    n_ev = jnp.sum(mf)
    inv_n = 1.0 / jnp.maximum(n_ev, 1.0)

    cntw = jnp.minimum(horizon, (_T - 1) - col)
    g = jnp.where(cntw > 0, mf / jnp.maximum(cntw, 1).astype(jnp.float32), 0.0)

    def shr(x, n):
        if n >= _T:
            return jnp.zeros_like(x)
        return jnp.concatenate([jnp.zeros((_B, n), dtype=x.dtype), x[:, :-n]], axis=1)

    big_g = g
    sh = 1
    while sh < _T:
        big_g = big_g + shr(big_g, sh)
        sh *= 2
    hp1 = jnp.minimum(horizon + 1, _T + 1)
    shifted = big_g
    for bit in range(12):
        amt = 1 << bit
        cond = ((hp1 >> bit) & 1) == 1
        shifted = jnp.where(cond, shr(shifted, amt), shifted)
    w_scr[...] = shr(big_g, 1) - shifted

    bsv = bs_ref[...]
    sidx = jax.lax.broadcasted_iota(jnp.int32, (_L, _TTL), 1)
    empty = bsv == jnp.int32(-1)
    first_empty = jnp.min(jnp.where(empty, sidx, _BIG), axis=1, keepdims=True)
    minval = jnp.min(bsv, axis=1, keepdims=True)
    first_min = jnp.min(jnp.where(bsv == minval, sidx, _BIG), axis=1, keepdims=True)
    slot = jnp.where(first_empty < _BIG, first_empty, first_min)

    age = jnp.maximum(step - bsv, 0).astype(jnp.float32)
    wgt = jnp.exp2(-age / _HALF_LIFE) * (bsv >= 0).astype(jnp.float32)
    wgt = jnp.where(sidx == slot, 0.0, wgt)
    wsum = jnp.sum(wgt, axis=1, keepdims=True) + 1.0
    wsn_scr[...] = wgt / wsum
    ls_scr[...] = inv_n / wsum


_NBUF = 3
_CHUNKS = 16
_CPL = _CHUNKS // _L


def _fused_kernel(scal_ref, p_ref, bs_ref, x_hbm, be_hbm, out_ref,
                  w_scr, wsn_scr, ls_scr, bank_scr,
                  b0, b1, b2, s0, s1, s2, sbank):
    bufs = [b0, b1, b2]
    sems = [s0, s1, s2]
    bank_cp = pltpu.async_copy(be_hbm, bank_scr, sbank)
    cps = {}
    for g in range(_NBUF):
        cps[g] = pltpu.async_copy(x_hbm.at[g], bufs[g], sems[g])

    _prep(scal_ref, p_ref, bs_ref, w_scr, wsn_scr, ls_scr)

    acc = jnp.zeros((1, _D), jnp.float32)
    for g in range(_CHUNKS):
        cps[g].wait()
        xb = bufs[g % _NBUF]
        for h in range(2):
            x = xb[pl.ds(h * _T, _T), :]
            wrow = w_scr[pl.ds((g % _CPL) * 2 + h, 1), :]
            acc += jax.lax.dot_general(
                wrow, x * x, (((1,), (0,)), ((), ())),
                preferred_element_type=jnp.float32)
        if g % _CPL == _CPL - 1:
            layer = g // _CPL
            if layer == 0:
                bank_cp.wait()
            bank = jax.lax.dot_general(
                wsn_scr[pl.ds(layer, 1), :], bank_scr[layer],
                (((1,), (0,)), ((), ())),
                preferred_element_type=jnp.float32)
            out_ref[pl.ds(layer, 1)] = (bank + ls_scr[pl.ds(layer, 1), :] * acc)[None]
            acc = jnp.zeros((1, _D), jnp.float32)
        if g + _NBUF < _CHUNKS:
            ng = g + _NBUF
            cps[ng] = pltpu.async_copy(x_hbm.at[ng], bufs[ng % _NBUF], sems[ng % _NBUF])


@jax.jit
def kernel(pressure, states, bank_evidence, bank_step, bank_event_count,
           baseline_future_energy, step, horizon_H, events_k):
    del bank_event_count, baseline_future_energy
    scal = jnp.stack([jnp.asarray(step, jnp.int32),
                      jnp.asarray(horizon_H, jnp.int32),
                      jnp.asarray(events_k, jnp.int32)])

    score = pl.pallas_call(
        _fused_kernel,
        in_specs=[
            pl.BlockSpec(memory_space=pltpu.MemorySpace.SMEM),
            pl.BlockSpec(memory_space=pltpu.MemorySpace.VMEM),
            pl.BlockSpec(memory_space=pltpu.MemorySpace.VMEM),
            pl.BlockSpec(memory_space=pltpu.MemorySpace.HBM),
            pl.BlockSpec(memory_space=pltpu.MemorySpace.HBM),
        ],
        out_specs=pl.BlockSpec(memory_space=pltpu.MemorySpace.VMEM),
        out_shape=jax.ShapeDtypeStruct((_L, 1, _D), jnp.float32),
        scratch_shapes=[
            pltpu.VMEM((_B, _T), jnp.float32),
            pltpu.VMEM((_L, _TTL), jnp.float32),
            pltpu.VMEM((_L, 1), jnp.float32),
            pltpu.VMEM((_L, _TTL, _D), jnp.float32),
            pltpu.VMEM((2 * _T, _D), jnp.float32),
            pltpu.VMEM((2 * _T, _D), jnp.float32),
            pltpu.VMEM((2 * _T, _D), jnp.float32),
            pltpu.SemaphoreType.DMA,
            pltpu.SemaphoreType.DMA,
            pltpu.SemaphoreType.DMA,
            pltpu.SemaphoreType.DMA,
        ],
    )(scal, pressure, bank_step,
      states.reshape(_CHUNKS, 2 * _T, _D), bank_evidence)

    return score.reshape(_L, _D)
